# baseline (device time: 32920 ns/iter reference)
import jax
import jax.numpy as jnp
from jax import lax
from jax.experimental import pallas as pl
from jax.experimental.pallas import tpu as pltpu

N_DEV = 32
N = 1024
M = 1024
M_PER = M // N_DEV
N_CLS = 16
N_GRP = 4
GRP = N_CLS // N_GRP


def _pos(t: int, r: int) -> int:
    z, y = r // 4, r % 4
    return 8 * z + 2 * y + (t ^ (y & 1))


def kernel(A, B):
    def body(a_ref, b_ref, out_ref,
             blk0_ref, blk1_ref, xrecv_ref, comb_ref, recv_ref,
             xsend_sems, xrecv_sems, send_sems, recv_sems):
        my = lax.axis_index("i")
        q = my % 8
        yy = q // 2
        t_my = (q % 2) ^ (yy % 2)
        my_rank = 4 * (my // 8) + yy
        partner = my ^ 1

        barrier_sem = pltpu.get_barrier_semaphore()
        pl.semaphore_signal(barrier_sem, inc=1, device_id=(partner,),
                            device_id_type=pl.DeviceIdType.MESH)
        for r in range(N_CLS):
            target = 8 * (r // 4) + 2 * (r % 4) + (t_my ^ ((r % 4) & 1))

            @pl.when(my_rank != r)
            def _(r=r, target=target):
                pl.semaphore_signal(barrier_sem, inc=1, device_id=(target,),
                                    device_id_type=pl.DeviceIdType.MESH)

        a16 = a_ref[...].astype(jnp.bfloat16)
        b16 = b_ref[...].astype(jnp.bfloat16)

        for g in range(N_GRP):
            sl = pl.ds(g * GRP, GRP)
            for t, blk in ((0, blk0_ref), (1, blk1_ref)):
                rows = jnp.concatenate(
                    [a16[_pos(t, r) * M_PER:(_pos(t, r) + 1) * M_PER, :]
                     for r in range(g * GRP, (g + 1) * GRP)], axis=0)
                blk[sl] = jnp.dot(
                    rows, b16, preferred_element_type=jnp.float32,
                ).astype(jnp.bfloat16).reshape(GRP, M_PER, N)

            if g == 0:
                pl.semaphore_wait(barrier_sem, N_CLS)

            @pl.when(t_my == 0)
            def _(sl=sl, g=g):
                pltpu.make_async_remote_copy(
                    src_ref=blk1_ref.at[sl], dst_ref=xrecv_ref.at[sl],
                    send_sem=xsend_sems.at[g], recv_sem=xrecv_sems.at[g],
                    device_id=(partner,), device_id_type=pl.DeviceIdType.MESH,
                ).start()

            @pl.when(t_my == 1)
            def _(sl=sl, g=g):
                pltpu.make_async_remote_copy(
                    src_ref=blk0_ref.at[sl], dst_ref=xrecv_ref.at[sl],
                    send_sem=xsend_sems.at[g], recv_sem=xrecv_sems.at[g],
                    device_id=(partner,), device_id_type=pl.DeviceIdType.MESH,
                ).start()

        for g in range(N_GRP):
            sl = pl.ds(g * GRP, GRP)
            pltpu.make_async_remote_copy(
                src_ref=blk0_ref.at[sl], dst_ref=xrecv_ref.at[sl],
                send_sem=xsend_sems.at[g], recv_sem=xrecv_sems.at[g],
                device_id=(partner,), device_id_type=pl.DeviceIdType.MESH,
            ).wait_recv()

            @pl.when(t_my == 0)
            def _(sl=sl):
                comb_ref[sl] = (
                    blk0_ref[sl].astype(jnp.float32)
                    + xrecv_ref[sl].astype(jnp.float32)
                ).astype(jnp.bfloat16)

            @pl.when(t_my == 1)
            def _(sl=sl):
                comb_ref[sl] = (
                    blk1_ref[sl].astype(jnp.float32)
                    + xrecv_ref[sl].astype(jnp.float32)
                ).astype(jnp.bfloat16)

            for r in range(g * GRP, (g + 1) * GRP):
                target = 8 * (r // 4) + 2 * (r % 4) + (t_my ^ ((r % 4) & 1))

                @pl.when(my_rank != r)
                def _(r=r, target=target):
                    pltpu.make_async_remote_copy(
                        src_ref=comb_ref.at[r],
                        dst_ref=recv_ref.at[my_rank],
                        send_sem=send_sems.at[r],
                        recv_sem=recv_sems.at[my_rank],
                        device_id=(target,), device_id_type=pl.DeviceIdType.MESH,
                    ).start()

        recv_ref[my_rank] = comb_ref[my_rank]

        for r in range(N_CLS):
            @pl.when(my_rank != r)
            def _(r=r):
                pltpu.make_async_remote_copy(
                    src_ref=comb_ref.at[r],
                    dst_ref=recv_ref.at[r],
                    send_sem=send_sems.at[r],
                    recv_sem=recv_sems.at[r],
                    device_id=(partner,), device_id_type=pl.DeviceIdType.MESH,
                ).wait_recv()

        out_ref[...] = jnp.sum(recv_ref[...].astype(jnp.float32), axis=0)

        for g in range(N_GRP):
            sl = pl.ds(g * GRP, GRP)
            pltpu.make_async_remote_copy(
                src_ref=blk0_ref.at[sl], dst_ref=xrecv_ref.at[sl],
                send_sem=xsend_sems.at[g], recv_sem=xrecv_sems.at[g],
                device_id=(partner,), device_id_type=pl.DeviceIdType.MESH,
            ).wait_send()
        for r in range(N_CLS):
            @pl.when(my_rank != r)
            def _(r=r):
                pltpu.make_async_remote_copy(
                    src_ref=comb_ref.at[r],
                    dst_ref=recv_ref.at[my_rank],
                    send_sem=send_sems.at[r],
                    recv_sem=recv_sems.at[my_rank],
                    device_id=(partner,), device_id_type=pl.DeviceIdType.MESH,
                ).wait_send()

    return pl.pallas_call(
        body,
        out_shape=jax.ShapeDtypeStruct((M_PER, N), jnp.float32),
        in_specs=[
            pl.BlockSpec(memory_space=pltpu.VMEM),
            pl.BlockSpec(memory_space=pltpu.VMEM),
        ],
        out_specs=pl.BlockSpec(memory_space=pltpu.VMEM),
        scratch_shapes=[
            pltpu.VMEM((N_CLS, M_PER, N), jnp.bfloat16),
            pltpu.VMEM((N_CLS, M_PER, N), jnp.bfloat16),
            pltpu.VMEM((N_CLS, M_PER, N), jnp.bfloat16),
            pltpu.VMEM((N_CLS, M_PER, N), jnp.bfloat16),
            pltpu.VMEM((N_CLS, M_PER, N), jnp.bfloat16),
            pltpu.SemaphoreType.DMA((N_GRP,)),
            pltpu.SemaphoreType.DMA((N_GRP,)),
            pltpu.SemaphoreType.DMA((N_CLS,)),
            pltpu.SemaphoreType.DMA((N_CLS,)),
        ],
        compiler_params=pltpu.CompilerParams(collective_id=0),
    )(A, B)


# device time: 32618 ns/iter; 1.0093x vs baseline; 1.0093x over previous
import jax
import jax.numpy as jnp
from jax import lax
from jax.experimental import pallas as pl
from jax.experimental.pallas import tpu as pltpu

N_DEV = 32
N = 1024
M = 1024
M_PER = M // N_DEV
N_CLS = 16


def _pos(t: int, r: int) -> int:
    z, y = r // 4, r % 4
    return 8 * z + 2 * y + (t ^ (y & 1))


def kernel(A, B):
    def body(a_ref, b_ref, out_ref,
             blk0_ref, blk1_ref, xrecv_ref, comb_ref, recv_ref,
             xsend_sem, xrecv_sem, send_sems, recv_sems):
        my = lax.axis_index("i")
        q = my % 8
        yy = q // 2
        t_my = (q % 2) ^ (yy % 2)
        my_rank = 4 * (my // 8) + yy
        partner = my ^ 1

        barrier_sem = pltpu.get_barrier_semaphore()
        pl.semaphore_signal(barrier_sem, inc=1, device_id=(partner,),
                            device_id_type=pl.DeviceIdType.MESH)
        for r in range(N_CLS):
            target = 8 * (r // 4) + 2 * (r % 4) + (t_my ^ ((r % 4) & 1))

            @pl.when(my_rank != r)
            def _(r=r, target=target):
                pl.semaphore_signal(barrier_sem, inc=1, device_id=(target,),
                                    device_id_type=pl.DeviceIdType.MESH)

        part = jnp.dot(
            a_ref[...].astype(jnp.bfloat16),
            b_ref[...].astype(jnp.bfloat16),
            preferred_element_type=jnp.float32,
        ).astype(jnp.bfloat16)

        for r in range(N_CLS):
            c0, c1 = _pos(0, r), _pos(1, r)
            blk0_ref[r] = part[c0 * M_PER:(c0 + 1) * M_PER, :]
            blk1_ref[r] = part[c1 * M_PER:(c1 + 1) * M_PER, :]

        pl.semaphore_wait(barrier_sem, N_CLS)

        for g in range(4):
            sl = pl.ds(g * 4, 4)

            @pl.when(t_my == 0)
            def _(sl=sl, g=g):
                pltpu.make_async_remote_copy(
                    src_ref=blk1_ref.at[sl], dst_ref=xrecv_ref.at[sl],
                    send_sem=xsend_sem.at[g], recv_sem=xrecv_sem.at[g],
                    device_id=(partner,), device_id_type=pl.DeviceIdType.MESH,
                ).start()

            @pl.when(t_my == 1)
            def _(sl=sl, g=g):
                pltpu.make_async_remote_copy(
                    src_ref=blk0_ref.at[sl], dst_ref=xrecv_ref.at[sl],
                    send_sem=xsend_sem.at[g], recv_sem=xrecv_sem.at[g],
                    device_id=(partner,), device_id_type=pl.DeviceIdType.MESH,
                ).start()

        for g in range(4):
            sl = pl.ds(g * 4, 4)
            pltpu.make_async_remote_copy(
                src_ref=blk0_ref.at[sl], dst_ref=xrecv_ref.at[sl],
                send_sem=xsend_sem.at[g], recv_sem=xrecv_sem.at[g],
                device_id=(partner,), device_id_type=pl.DeviceIdType.MESH,
            ).wait_recv()

            @pl.when(t_my == 0)
            def _(sl=sl):
                comb_ref[sl] = (
                    blk0_ref[sl].astype(jnp.float32)
                    + xrecv_ref[sl].astype(jnp.float32)
                ).astype(jnp.bfloat16)

            @pl.when(t_my == 1)
            def _(sl=sl):
                comb_ref[sl] = (
                    blk1_ref[sl].astype(jnp.float32)
                    + xrecv_ref[sl].astype(jnp.float32)
                ).astype(jnp.bfloat16)

            for r in range(g * 4, (g + 1) * 4):
                target = 8 * (r // 4) + 2 * (r % 4) + (t_my ^ ((r % 4) & 1))

                @pl.when(my_rank != r)
                def _(r=r, target=target):
                    pltpu.make_async_remote_copy(
                        src_ref=comb_ref.at[r],
                        dst_ref=recv_ref.at[my_rank],
                        send_sem=send_sems.at[r],
                        recv_sem=recv_sems.at[my_rank],
                        device_id=(target,), device_id_type=pl.DeviceIdType.MESH,
                    ).start()

        recv_ref[my_rank] = comb_ref[my_rank]

        for r in range(N_CLS):
            @pl.when(my_rank != r)
            def _(r=r):
                pltpu.make_async_remote_copy(
                    src_ref=comb_ref.at[r],
                    dst_ref=recv_ref.at[r],
                    send_sem=send_sems.at[r],
                    recv_sem=recv_sems.at[r],
                    device_id=(partner,), device_id_type=pl.DeviceIdType.MESH,
                ).wait_recv()

        out_ref[...] = jnp.sum(recv_ref[...].astype(jnp.float32), axis=0)

        for g in range(4):
            sl = pl.ds(g * 4, 4)
            pltpu.make_async_remote_copy(
                src_ref=blk0_ref.at[sl], dst_ref=xrecv_ref.at[sl],
                send_sem=xsend_sem.at[g], recv_sem=xrecv_sem.at[g],
                device_id=(partner,), device_id_type=pl.DeviceIdType.MESH,
            ).wait_send()
        for r in range(N_CLS):
            @pl.when(my_rank != r)
            def _(r=r):
                pltpu.make_async_remote_copy(
                    src_ref=comb_ref.at[r],
                    dst_ref=recv_ref.at[my_rank],
                    send_sem=send_sems.at[r],
                    recv_sem=recv_sems.at[my_rank],
                    device_id=(partner,), device_id_type=pl.DeviceIdType.MESH,
                ).wait_send()

    return pl.pallas_call(
        body,
        out_shape=jax.ShapeDtypeStruct((M_PER, N), jnp.float32),
        in_specs=[
            pl.BlockSpec(memory_space=pltpu.VMEM),
            pl.BlockSpec(memory_space=pltpu.VMEM),
        ],
        out_specs=pl.BlockSpec(memory_space=pltpu.VMEM),
        scratch_shapes=[
            pltpu.VMEM((N_CLS, M_PER, N), jnp.bfloat16),
            pltpu.VMEM((N_CLS, M_PER, N), jnp.bfloat16),
            pltpu.VMEM((N_CLS, M_PER, N), jnp.bfloat16),
            pltpu.VMEM((N_CLS, M_PER, N), jnp.bfloat16),
            pltpu.VMEM((N_CLS, M_PER, N), jnp.bfloat16),
            pltpu.SemaphoreType.DMA((4,)),
            pltpu.SemaphoreType.DMA((4,)),
            pltpu.SemaphoreType.DMA((N_CLS,)),
            pltpu.SemaphoreType.DMA((N_CLS,)),
        ],
        compiler_params=pltpu.CompilerParams(collective_id=0),
    )(A, B)


# device time: 31372 ns/iter; 1.0493x vs baseline; 1.0397x over previous
import jax
import jax.numpy as jnp
from jax import lax
from jax.experimental import pallas as pl
from jax.experimental.pallas import tpu as pltpu

N_DEV = 32
N = 1024
M = 1024
M_PER = M // N_DEV
N_CLS = 16


def _pos(t: int, r: int) -> int:
    z, y = r // 4, r % 4
    return 8 * z + 2 * y + (t ^ (y & 1))


def kernel(A, B):
    def body(a_ref, b_ref, out_ref,
             blk0_ref, blk1_ref, xrecv_ref, comb_ref, recv_ref,
             xsend_sem, xrecv_sem, send_sems, recv_sems):
        my = lax.axis_index("i")
        q = my % 8
        yy = q // 2
        t_my = (q % 2) ^ (yy % 2)
        my_rank = 4 * (my // 8) + yy
        partner = my ^ 1

        barrier_sem = pltpu.get_barrier_semaphore()
        pl.semaphore_signal(barrier_sem, inc=1, device_id=(partner,),
                            device_id_type=pl.DeviceIdType.MESH)
        for r in range(N_CLS):
            target = 8 * (r // 4) + 2 * (r % 4) + (t_my ^ ((r % 4) & 1))

            @pl.when(my_rank != r)
            def _(r=r, target=target):
                pl.semaphore_signal(barrier_sem, inc=1, device_id=(target,),
                                    device_id_type=pl.DeviceIdType.MESH)

        part = jnp.dot(
            a_ref[...].astype(jnp.bfloat16),
            b_ref[...].astype(jnp.bfloat16),
            preferred_element_type=jnp.float32,
        ).astype(jnp.bfloat16)

        for r in range(N_CLS):
            c0, c1 = _pos(0, r), _pos(1, r)
            blk0_ref[r] = part[c0 * M_PER:(c0 + 1) * M_PER, :]
            blk1_ref[r] = part[c1 * M_PER:(c1 + 1) * M_PER, :]

        pl.semaphore_wait(barrier_sem, N_CLS)

        for g in range(8):
            sl = pl.ds(g * 2, 2)

            @pl.when(t_my == 0)
            def _(sl=sl, g=g):
                pltpu.make_async_remote_copy(
                    src_ref=blk1_ref.at[sl], dst_ref=xrecv_ref.at[sl],
                    send_sem=xsend_sem.at[g], recv_sem=xrecv_sem.at[g],
                    device_id=(partner,), device_id_type=pl.DeviceIdType.MESH,
                ).start()

            @pl.when(t_my == 1)
            def _(sl=sl, g=g):
                pltpu.make_async_remote_copy(
                    src_ref=blk0_ref.at[sl], dst_ref=xrecv_ref.at[sl],
                    send_sem=xsend_sem.at[g], recv_sem=xrecv_sem.at[g],
                    device_id=(partner,), device_id_type=pl.DeviceIdType.MESH,
                ).start()

        for g in range(8):
            sl = pl.ds(g * 2, 2)
            pltpu.make_async_remote_copy(
                src_ref=blk0_ref.at[sl], dst_ref=xrecv_ref.at[sl],
                send_sem=xsend_sem.at[g], recv_sem=xrecv_sem.at[g],
                device_id=(partner,), device_id_type=pl.DeviceIdType.MESH,
            ).wait_recv()

            @pl.when(t_my == 0)
            def _(sl=sl):
                comb_ref[sl] = (
                    blk0_ref[sl].astype(jnp.float32)
                    + xrecv_ref[sl].astype(jnp.float32)
                ).astype(jnp.bfloat16)

            @pl.when(t_my == 1)
            def _(sl=sl):
                comb_ref[sl] = (
                    blk1_ref[sl].astype(jnp.float32)
                    + xrecv_ref[sl].astype(jnp.float32)
                ).astype(jnp.bfloat16)

            for r in range(g * 2, (g + 1) * 2):
                target = 8 * (r // 4) + 2 * (r % 4) + (t_my ^ ((r % 4) & 1))

                @pl.when(my_rank != r)
                def _(r=r, target=target):
                    pltpu.make_async_remote_copy(
                        src_ref=comb_ref.at[r],
                        dst_ref=recv_ref.at[my_rank],
                        send_sem=send_sems.at[r],
                        recv_sem=recv_sems.at[my_rank],
                        device_id=(target,), device_id_type=pl.DeviceIdType.MESH,
                    ).start()

        recv_ref[my_rank] = comb_ref[my_rank]

        for r in range(N_CLS):
            @pl.when(my_rank != r)
            def _(r=r):
                pltpu.make_async_remote_copy(
                    src_ref=comb_ref.at[r],
                    dst_ref=recv_ref.at[r],
                    send_sem=send_sems.at[r],
                    recv_sem=recv_sems.at[r],
                    device_id=(partner,), device_id_type=pl.DeviceIdType.MESH,
                ).wait_recv()

        out_ref[...] = jnp.sum(recv_ref[...].astype(jnp.float32), axis=0)

        for g in range(8):
            sl = pl.ds(g * 2, 2)
            pltpu.make_async_remote_copy(
                src_ref=blk0_ref.at[sl], dst_ref=xrecv_ref.at[sl],
                send_sem=xsend_sem.at[g], recv_sem=xrecv_sem.at[g],
                device_id=(partner,), device_id_type=pl.DeviceIdType.MESH,
            ).wait_send()
        for r in range(N_CLS):
            @pl.when(my_rank != r)
            def _(r=r):
                pltpu.make_async_remote_copy(
                    src_ref=comb_ref.at[r],
                    dst_ref=recv_ref.at[my_rank],
                    send_sem=send_sems.at[r],
                    recv_sem=recv_sems.at[my_rank],
                    device_id=(partner,), device_id_type=pl.DeviceIdType.MESH,
                ).wait_send()

    return pl.pallas_call(
        body,
        out_shape=jax.ShapeDtypeStruct((M_PER, N), jnp.float32),
        in_specs=[
            pl.BlockSpec(memory_space=pltpu.VMEM),
            pl.BlockSpec(memory_space=pltpu.VMEM),
        ],
        out_specs=pl.BlockSpec(memory_space=pltpu.VMEM),
        scratch_shapes=[
            pltpu.VMEM((N_CLS, M_PER, N), jnp.bfloat16),
            pltpu.VMEM((N_CLS, M_PER, N), jnp.bfloat16),
            pltpu.VMEM((N_CLS, M_PER, N), jnp.bfloat16),
            pltpu.VMEM((N_CLS, M_PER, N), jnp.bfloat16),
            pltpu.VMEM((N_CLS, M_PER, N), jnp.bfloat16),
            pltpu.SemaphoreType.DMA((8,)),
            pltpu.SemaphoreType.DMA((8,)),
            pltpu.SemaphoreType.DMA((N_CLS,)),
            pltpu.SemaphoreType.DMA((N_CLS,)),
        ],
        compiler_params=pltpu.CompilerParams(collective_id=0),
    )(A, B)


# device time: 30769 ns/iter; 1.0699x vs baseline; 1.0196x over previous
import jax
import jax.numpy as jnp
from jax import lax
from jax.experimental import pallas as pl
from jax.experimental.pallas import tpu as pltpu

N_DEV = 32
N = 1024
M = 1024
M_PER = M // N_DEV
N_CLS = 16


def _pos(t: int, r: int) -> int:
    z, y = r // 4, r % 4
    return 8 * z + 2 * y + (t ^ (y & 1))


def kernel(A, B):
    def body(a_ref, b_ref, out_ref,
             blk0_ref, blk1_ref, xrecv_ref, comb_ref, recv_ref,
             xsend_sem, xrecv_sem, send_sems, recv_sems):
        my = lax.axis_index("i")
        q = my % 8
        yy = q // 2
        t_my = (q % 2) ^ (yy % 2)
        my_rank = 4 * (my // 8) + yy
        partner = my ^ 1

        barrier_sem = pltpu.get_barrier_semaphore()
        pl.semaphore_signal(barrier_sem, inc=1, device_id=(partner,),
                            device_id_type=pl.DeviceIdType.MESH)
        for r in range(N_CLS):
            target = 8 * (r // 4) + 2 * (r % 4) + (t_my ^ ((r % 4) & 1))

            @pl.when(my_rank != r)
            def _(r=r, target=target):
                pl.semaphore_signal(barrier_sem, inc=1, device_id=(target,),
                                    device_id_type=pl.DeviceIdType.MESH)

        part = jnp.dot(
            a_ref[...].astype(jnp.bfloat16),
            b_ref[...].astype(jnp.bfloat16),
            preferred_element_type=jnp.float32,
        ).astype(jnp.bfloat16)

        for r in range(N_CLS):
            c0, c1 = _pos(0, r), _pos(1, r)
            blk0_ref[r] = part[c0 * M_PER:(c0 + 1) * M_PER, :]
            blk1_ref[r] = part[c1 * M_PER:(c1 + 1) * M_PER, :]

        pl.semaphore_wait(barrier_sem, N_CLS)

        for g in range(16):
            sl = pl.ds(g * 1, 1)

            @pl.when(t_my == 0)
            def _(sl=sl, g=g):
                pltpu.make_async_remote_copy(
                    src_ref=blk1_ref.at[sl], dst_ref=xrecv_ref.at[sl],
                    send_sem=xsend_sem.at[g], recv_sem=xrecv_sem.at[g],
                    device_id=(partner,), device_id_type=pl.DeviceIdType.MESH,
                ).start()

            @pl.when(t_my == 1)
            def _(sl=sl, g=g):
                pltpu.make_async_remote_copy(
                    src_ref=blk0_ref.at[sl], dst_ref=xrecv_ref.at[sl],
                    send_sem=xsend_sem.at[g], recv_sem=xrecv_sem.at[g],
                    device_id=(partner,), device_id_type=pl.DeviceIdType.MESH,
                ).start()

        for g in range(16):
            sl = pl.ds(g * 1, 1)
            pltpu.make_async_remote_copy(
                src_ref=blk0_ref.at[sl], dst_ref=xrecv_ref.at[sl],
                send_sem=xsend_sem.at[g], recv_sem=xrecv_sem.at[g],
                device_id=(partner,), device_id_type=pl.DeviceIdType.MESH,
            ).wait_recv()

            @pl.when(t_my == 0)
            def _(sl=sl):
                comb_ref[sl] = (
                    blk0_ref[sl].astype(jnp.float32)
                    + xrecv_ref[sl].astype(jnp.float32)
                ).astype(jnp.bfloat16)

            @pl.when(t_my == 1)
            def _(sl=sl):
                comb_ref[sl] = (
                    blk1_ref[sl].astype(jnp.float32)
                    + xrecv_ref[sl].astype(jnp.float32)
                ).astype(jnp.bfloat16)

            for r in range(g * 1, (g + 1) * 1):
                target = 8 * (r // 4) + 2 * (r % 4) + (t_my ^ ((r % 4) & 1))

                @pl.when(my_rank != r)
                def _(r=r, target=target):
                    pltpu.make_async_remote_copy(
                        src_ref=comb_ref.at[r],
                        dst_ref=recv_ref.at[my_rank],
                        send_sem=send_sems.at[r],
                        recv_sem=recv_sems.at[my_rank],
                        device_id=(target,), device_id_type=pl.DeviceIdType.MESH,
                    ).start()

        recv_ref[my_rank] = comb_ref[my_rank]

        for r in range(N_CLS):
            @pl.when(my_rank != r)
            def _(r=r):
                pltpu.make_async_remote_copy(
                    src_ref=comb_ref.at[r],
                    dst_ref=recv_ref.at[r],
                    send_sem=send_sems.at[r],
                    recv_sem=recv_sems.at[r],
                    device_id=(partner,), device_id_type=pl.DeviceIdType.MESH,
                ).wait_recv()

        out_ref[...] = jnp.sum(recv_ref[...].astype(jnp.float32), axis=0)

        for g in range(16):
            sl = pl.ds(g * 1, 1)
            pltpu.make_async_remote_copy(
                src_ref=blk0_ref.at[sl], dst_ref=xrecv_ref.at[sl],
                send_sem=xsend_sem.at[g], recv_sem=xrecv_sem.at[g],
                device_id=(partner,), device_id_type=pl.DeviceIdType.MESH,
            ).wait_send()
        for r in range(N_CLS):
            @pl.when(my_rank != r)
            def _(r=r):
                pltpu.make_async_remote_copy(
                    src_ref=comb_ref.at[r],
                    dst_ref=recv_ref.at[my_rank],
                    send_sem=send_sems.at[r],
                    recv_sem=recv_sems.at[my_rank],
                    device_id=(partner,), device_id_type=pl.DeviceIdType.MESH,
                ).wait_send()

    return pl.pallas_call(
        body,
        out_shape=jax.ShapeDtypeStruct((M_PER, N), jnp.float32),
        in_specs=[
            pl.BlockSpec(memory_space=pltpu.VMEM),
            pl.BlockSpec(memory_space=pltpu.VMEM),
        ],
        out_specs=pl.BlockSpec(memory_space=pltpu.VMEM),
        scratch_shapes=[
            pltpu.VMEM((N_CLS, M_PER, N), jnp.bfloat16),
            pltpu.VMEM((N_CLS, M_PER, N), jnp.bfloat16),
            pltpu.VMEM((N_CLS, M_PER, N), jnp.bfloat16),
            pltpu.VMEM((N_CLS, M_PER, N), jnp.bfloat16),
            pltpu.VMEM((N_CLS, M_PER, N), jnp.bfloat16),
            pltpu.SemaphoreType.DMA((16,)),
            pltpu.SemaphoreType.DMA((16,)),
            pltpu.SemaphoreType.DMA((N_CLS,)),
            pltpu.SemaphoreType.DMA((N_CLS,)),
        ],
        compiler_params=pltpu.CompilerParams(collective_id=0),
    )(A, B)
